# baseline (speedup 1.0000x reference)
import jax
import jax.numpy as jnp
from jax.experimental import pallas as pl
from jax.experimental.pallas import tpu as pltpu

_D = 64
_C = 1024
_B = 16
_T = 1024
_N = _B * _T
_U = 8


def _vq_body(x_ref, e_ref, q_ref, idx_ref, pplx_ref, counts_ref):
    b = pl.program_id(0)
    e = e_ref[...]
    e_bf = e.astype(jnp.bfloat16)
    em2_bf = e_bf * jnp.bfloat16(-2.0)
    e2 = jnp.sum(e * e, axis=0)
    e2_col = e2[None, :].T

    @pl.when(b == 0)
    def _init():
        counts_ref[...] = jnp.zeros_like(counts_ref)

    oh_acc = jnp.zeros((_C, _T), jnp.bfloat16)
    for i in range(_U):
        xb = x_ref[i]
        xem2 = jax.lax.dot_general(em2_bf, xb.astype(jnp.bfloat16),
                                   (((0,), (0,)), ((), ())),
                                   preferred_element_type=jnp.float32)
        x2 = jnp.sum(xb * xb, axis=0)
        dist = (x2[None, :] + xem2) + e2_col
        idx = jnp.argmin(dist, axis=0).astype(jnp.int32)
        idx_ref[i] = idx
        oh_bf = (jax.lax.broadcasted_iota(jnp.int32, (_C, _T), 0)
                 == idx[None, :]).astype(jnp.bfloat16)
        q = jax.lax.dot_general(e_bf, oh_bf,
                                (((1,), (0,)), ((), ())),
                                preferred_element_type=jnp.float32)
        q_ref[i] = xb + (q - xb)
        oh_acc = oh_acc + oh_bf

    counts_ref[...] += jnp.sum(oh_acc.astype(jnp.float32), axis=1, keepdims=True)

    @pl.when(b == (_B // _U) - 1)
    def _fin():
        probs = counts_ref[...] * (1.0 / _N)
        ent = -jnp.sum(probs * jnp.log(probs + 1e-10))
        pplx_ref[...] = jnp.exp(ent).reshape(1, 1)


def _make_call(interpret=False):
    return pl.pallas_call(
        _vq_body,
        grid=(_B // _U,),
        in_specs=[
            pl.BlockSpec((_U, _D, _T), lambda b: (b, 0, 0)),
            pl.BlockSpec((_D, _C), lambda b: (0, 0)),
        ],
        out_specs=[
            pl.BlockSpec((_U, _D, _T), lambda b: (b, 0, 0)),
            pl.BlockSpec((_U, _T), lambda b: (b, 0)),
            pl.BlockSpec((1, 1), lambda b: (0, 0)),
        ],
        out_shape=[
            jax.ShapeDtypeStruct((_B, _D, _T), jnp.float32),
            jax.ShapeDtypeStruct((_B, _T), jnp.int32),
            jax.ShapeDtypeStruct((1, 1), jnp.float32),
        ],
        scratch_shapes=[pltpu.VMEM((_C, 1), jnp.float32)],
        interpret=interpret,
    )


def kernel(x, embed):
    q, idx, pplx = _make_call()(x, embed)
    return q, idx, pplx[0, 0]


# speedup vs baseline: 1.0405x; 1.0405x over previous
import jax
import jax.numpy as jnp
from jax.experimental import pallas as pl
from jax.experimental.pallas import tpu as pltpu

_D = 64
_C = 1024
_B = 16
_T = 1024
_N = _B * _T
_U = 8


def _vq_body(x_ref, e_ref, q_ref, idx_ref, pplx_ref, counts_ref):
    b = pl.program_id(0)
    e = e_ref[...]
    e_bf = e.astype(jnp.bfloat16)
    em2_bf = e_bf * jnp.bfloat16(-2.0)
    e2 = jnp.sum(e * e, axis=0)
    e2_col = e2[None, :].T

    @pl.when(b == 0)
    def _init():
        counts_ref[...] = jnp.zeros_like(counts_ref)

    cnt = jnp.zeros((_C, 1), jnp.float32)
    for i in range(_U):
        xb = x_ref[i]
        xem2 = jax.lax.dot_general(em2_bf, xb.astype(jnp.bfloat16),
                                   (((0,), (0,)), ((), ())),
                                   preferred_element_type=jnp.float32)
        x2 = jnp.sum(xb * xb, axis=0)
        dist = (x2[None, :] + xem2) + e2_col
        idx = jnp.argmin(dist, axis=0).astype(jnp.int32)
        idx_ref[i] = idx
        oh_bf = (jax.lax.broadcasted_iota(jnp.int32, (_C, _T), 0)
                 == idx[None, :]).astype(jnp.bfloat16)
        q = jax.lax.dot_general(e_bf, oh_bf,
                                (((1,), (0,)), ((), ())),
                                preferred_element_type=jnp.float32)
        q_ref[i] = xb + (q - xb)
        cnt = cnt + jnp.sum(oh_bf.astype(jnp.float32), axis=1, keepdims=True)

    counts_ref[...] += cnt

    @pl.when(b == (_B // _U) - 1)
    def _fin():
        probs = counts_ref[...] * (1.0 / _N)
        ent = -jnp.sum(probs * jnp.log(probs + 1e-10))
        pplx_ref[...] = jnp.exp(ent).reshape(1, 1)


def _make_call(interpret=False):
    return pl.pallas_call(
        _vq_body,
        grid=(_B // _U,),
        in_specs=[
            pl.BlockSpec((_U, _D, _T), lambda b: (b, 0, 0)),
            pl.BlockSpec((_D, _C), lambda b: (0, 0)),
        ],
        out_specs=[
            pl.BlockSpec((_U, _D, _T), lambda b: (b, 0, 0)),
            pl.BlockSpec((_U, _T), lambda b: (b, 0)),
            pl.BlockSpec((1, 1), lambda b: (0, 0)),
        ],
        out_shape=[
            jax.ShapeDtypeStruct((_B, _D, _T), jnp.float32),
            jax.ShapeDtypeStruct((_B, _T), jnp.int32),
            jax.ShapeDtypeStruct((1, 1), jnp.float32),
        ],
        scratch_shapes=[pltpu.VMEM((_C, 1), jnp.float32)],
        interpret=interpret,
    )


def kernel(x, embed):
    q, idx, pplx = _make_call()(x, embed)
    return q, idx, pplx[0, 0]
